# baseline (speedup 1.0000x reference)
import functools

import jax
import jax.numpy as jnp
from jax import lax
from jax.experimental import pallas as pl
from jax.experimental.pallas import tpu as pltpu
from jax.experimental.pallas import tpu_sc as plsc

NUM_CORES = 2
NUM_SUBCORES = 16
NUM_WORKERS = NUM_CORES * NUM_SUBCORES
CHUNK = 80
NSLOT = 5


def _edge_embed_kernel(B, C, pc_hbm, rc_hbm, pm_hbm, num_hbm, ce_hbm, pe_hbm,
                       out_hbm, ce_v, pe_v, *bufs):
    per_w = B // NUM_WORKERS
    nchunks = per_w // C
    wid = lax.axis_index("s") * NUM_CORES + lax.axis_index("c")
    w0 = wid * per_w
    buf_slots = [bufs[5 * b:5 * b + 5] for b in range(NSLOT)]

    pl.run_scoped(
        functools.partial(_pipeline, C, per_w, nchunks, w0, buf_slots,
                          pc_hbm, rc_hbm, pm_hbm, num_hbm, ce_hbm, pe_hbm,
                          out_hbm, ce_v, pe_v),
        sems=pltpu.SemaphoreType.DMA((NSLOT, 5)),
    )


def _pipeline(C, per_w, nchunks, w0, buf_slots, pc_hbm, rc_hbm, pm_hbm,
              num_hbm, ce_hbm, pe_hbm, out_hbm, ce_v, pe_v, sems):
    pltpu.sync_copy(ce_hbm, ce_v)
    pltpu.sync_copy(pe_hbm, pe_v)

    def issue_in(i, b):
        pc_i, rc_i, pm_i, num_v, out_v = buf_slots[b]
        base = w0 + i * C
        pltpu.async_copy(pc_hbm.at[pl.ds(base, C)], pc_i, sems.at[b, 0])
        pltpu.async_copy(rc_hbm.at[pl.ds(base, C)], rc_i, sems.at[b, 1])
        pltpu.async_copy(pm_hbm.at[pl.ds(base, C)], pm_i, sems.at[b, 2])
        pltpu.async_copy(num_hbm.at[pl.ds(base, C)], num_v, sems.at[b, 3])

    def wait_in(i, b):
        pc_i, rc_i, pm_i, num_v, out_v = buf_slots[b]
        base = w0 + i * C
        pltpu.make_async_copy(pc_hbm.at[pl.ds(base, C)], pc_i,
                              sems.at[b, 0]).wait()
        pltpu.make_async_copy(rc_hbm.at[pl.ds(base, C)], rc_i,
                              sems.at[b, 1]).wait()
        pltpu.make_async_copy(pm_hbm.at[pl.ds(base, C)], pm_i,
                              sems.at[b, 2]).wait()
        pltpu.make_async_copy(num_hbm.at[pl.ds(base, C)], num_v,
                              sems.at[b, 3]).wait()

    def issue_out(i, b):
        pc_i, rc_i, pm_i, num_v, out_v = buf_slots[b]
        base = w0 + i * C
        pltpu.async_copy(out_v, out_hbm.at[pl.ds(base, C)], sems.at[b, 4])

    def wait_out(i, b):
        pc_i, rc_i, pm_i, num_v, out_v = buf_slots[b]
        base = w0 + i * C
        pltpu.make_async_copy(out_v, out_hbm.at[pl.ds(base, C)],
                              sems.at[b, 4]).wait()

    for b in range(NSLOT):
        issue_in(b, b)

    def super_body(k, carry):
        for b in range(NSLOT):
            i = NSLOT * k + b
            pc_i, rc_i, pm_i, num_v, out_v = buf_slots[b]
            wait_in(i, b)

            @pl.when(k > 0)
            def _():
                wait_out(i - NSLOT, b)

            iota16 = lax.iota(jnp.int32, 16)

            def assemble(g, carry2):
                e0 = g * 16
                lanes = e0 + iota16
                pcv = pc_i[pl.ds(e0, 16)] * 16
                rcv = rc_i[pl.ds(e0, 16)] * 16
                pmv = pm_i[pl.ds(e0, 16)] * 16
                for j in range(16):
                    v = plsc.load_gather(ce_v, [pcv + j])
                    plsc.store_scatter(
                        out_v, [lanes, jnp.full((16,), j, jnp.int32)], v)
                    v = plsc.load_gather(ce_v, [rcv + j])
                    plsc.store_scatter(
                        out_v, [lanes, jnp.full((16,), 16 + j, jnp.int32)], v)
                for j in range(8):
                    v = plsc.load_gather(pe_v, [pmv + j])
                    plsc.store_scatter(
                        out_v, [lanes, jnp.full((16,), 32 + j, jnp.int32)], v)
                for j in range(16):
                    e = e0 + j
                    out_v[e, pl.ds(40, 16)] = num_v[e, :]
                return carry2

            lax.fori_loop(0, C // 16, assemble, 0)

            @pl.when(i + NSLOT < nchunks)
            def _():
                issue_in(i + NSLOT, b)

            issue_out(i, b)
        return carry

    lax.fori_loop(0, nchunks // NSLOT, super_body, 0)
    for b in range(NSLOT):
        wait_out(nchunks - NSLOT + b, b)


def kernel(payment_curr, receiving_curr, payment_method, numericals,
           currencies_embed, payment_embed):
    B = payment_curr.shape[0]
    D_out = (2 * currencies_embed.shape[1] + payment_embed.shape[1]
             + numericals.shape[1])
    pe_pad = jnp.pad(payment_embed,
                     ((0, 0), (0, 16 - payment_embed.shape[1])))
    ce_flat = jnp.reshape(currencies_embed, (-1,))
    pe_flat = jnp.reshape(pe_pad, (-1,))
    mesh = plsc.VectorSubcoreMesh(core_axis_name="c", subcore_axis_name="s",
                                  num_cores=NUM_CORES)
    k = functools.partial(_edge_embed_kernel, B, CHUNK)
    slot_types = []
    for _ in range(NSLOT):
        slot_types += [
            pltpu.VMEM((CHUNK,), jnp.int32),
            pltpu.VMEM((CHUNK,), jnp.int32),
            pltpu.VMEM((CHUNK,), jnp.int32),
            pltpu.VMEM((CHUNK, 16), jnp.float32),
            pltpu.VMEM((CHUNK, 56), jnp.float32),
        ]
    run = pl.kernel(
        k,
        out_type=jax.ShapeDtypeStruct((B, D_out), jnp.float32),
        mesh=mesh,
        compiler_params=pltpu.CompilerParams(needs_layout_passes=False),
        scratch_types=[
            pltpu.VMEM((ce_flat.shape[0],), jnp.float32),
            pltpu.VMEM((pe_flat.shape[0],), jnp.float32),
        ] + slot_types,
    )
    return run(payment_curr, receiving_curr, payment_method, numericals,
               ce_flat, pe_flat)


# speedup vs baseline: 1.9396x; 1.9396x over previous
import functools

import jax
import jax.numpy as jnp
from jax import lax
from jax.experimental import pallas as pl
from jax.experimental.pallas import tpu as pltpu
from jax.experimental.pallas import tpu_sc as plsc

NUM_CORES = 2
NUM_SUBCORES = 16
NUM_WORKERS = NUM_CORES * NUM_SUBCORES
CHUNK = 80
NSLOT = 5


def _edge_embed_kernel(B, C, pc_hbm, rc_hbm, pm_hbm, num_hbm, ce_hbm, pe_hbm,
                       out_hbm, ce_v, pe_v, *bufs):
    per_w = B // NUM_WORKERS
    nchunks = per_w // C
    wid = lax.axis_index("s") * NUM_CORES + lax.axis_index("c")
    w0 = wid * per_w
    buf_slots = [bufs[5 * b:5 * b + 5] for b in range(NSLOT)]

    pl.run_scoped(
        functools.partial(_pipeline, C, per_w, nchunks, w0, buf_slots,
                          pc_hbm, rc_hbm, pm_hbm, num_hbm, ce_hbm, pe_hbm,
                          out_hbm, ce_v, pe_v),
        sems=pltpu.SemaphoreType.DMA((NSLOT, 5)),
    )


def _pipeline(C, per_w, nchunks, w0, buf_slots, pc_hbm, rc_hbm, pm_hbm,
              num_hbm, ce_hbm, pe_hbm, out_hbm, ce_v, pe_v, sems):
    pltpu.sync_copy(ce_hbm, ce_v)
    pltpu.sync_copy(pe_hbm, pe_v)

    def issue_in(i, b):
        pc_i, rc_i, pm_i, num_v, out_v = buf_slots[b]
        base = w0 + i * C
        pltpu.async_copy(pc_hbm.at[pl.ds(base, C)], pc_i, sems.at[b, 0])
        pltpu.async_copy(rc_hbm.at[pl.ds(base, C)], rc_i, sems.at[b, 1])
        pltpu.async_copy(pm_hbm.at[pl.ds(base, C)], pm_i, sems.at[b, 2])
        pltpu.async_copy(num_hbm.at[pl.ds(base, C)], num_v, sems.at[b, 3])

    def wait_in(i, b):
        pc_i, rc_i, pm_i, num_v, out_v = buf_slots[b]
        base = w0 + i * C
        pltpu.make_async_copy(pc_hbm.at[pl.ds(base, C)], pc_i,
                              sems.at[b, 0]).wait()
        pltpu.make_async_copy(rc_hbm.at[pl.ds(base, C)], rc_i,
                              sems.at[b, 1]).wait()
        pltpu.make_async_copy(pm_hbm.at[pl.ds(base, C)], pm_i,
                              sems.at[b, 2]).wait()
        pltpu.make_async_copy(num_hbm.at[pl.ds(base, C)], num_v,
                              sems.at[b, 3]).wait()

    def issue_out(i, b):
        pc_i, rc_i, pm_i, num_v, out_v = buf_slots[b]
        base = w0 + i * C
        pltpu.async_copy(out_v, out_hbm.at[pl.ds(base, C)], sems.at[b, 4])

    def wait_out(i, b):
        pc_i, rc_i, pm_i, num_v, out_v = buf_slots[b]
        base = w0 + i * C
        pltpu.make_async_copy(out_v, out_hbm.at[pl.ds(base, C)],
                              sems.at[b, 4]).wait()

    for b in range(NSLOT):
        issue_in(b, b)

    def super_body(k, carry):
        for b in range(NSLOT):
            i = NSLOT * k + b
            pc_i, rc_i, pm_i, num_v, out_v = buf_slots[b]
            wait_in(i, b)

            @pl.when(k > 0)
            def _():
                wait_out(i - NSLOT, b)

            def assemble(g, carry2):
                e0 = g * 16
                pcv = pc_i[pl.ds(e0, 16)] * 16
                rcv = rc_i[pl.ds(e0, 16)] * 16
                pmv = pm_i[pl.ds(e0, 16)] * 16
                for j in range(0, 16, 4):
                    vals = []
                    for t in range(4):
                        e = e0 + j + t
                        vals.append((ce_v[pl.ds(pcv[j + t], 16)],
                                     ce_v[pl.ds(rcv[j + t], 16)],
                                     pe_v[pl.ds(pmv[j + t], 16)],
                                     num_v[e, :]))
                    for t in range(4):
                        e = e0 + j + t
                        a, b, c, d = vals[t]
                        out_v[e, pl.ds(0, 16)] = a
                        out_v[e, pl.ds(16, 16)] = b
                        out_v[e, pl.ds(32, 16)] = c
                        out_v[e, pl.ds(40, 16)] = d
                return carry2

            lax.fori_loop(0, C // 16, assemble, 0)

            @pl.when(i + NSLOT < nchunks)
            def _():
                issue_in(i + NSLOT, b)

            issue_out(i, b)
        return carry

    lax.fori_loop(0, nchunks // NSLOT, super_body, 0)
    for b in range(NSLOT):
        wait_out(nchunks - NSLOT + b, b)


def kernel(payment_curr, receiving_curr, payment_method, numericals,
           currencies_embed, payment_embed):
    B = payment_curr.shape[0]
    D_out = (2 * currencies_embed.shape[1] + payment_embed.shape[1]
             + numericals.shape[1])
    pe_pad = jnp.pad(payment_embed,
                     ((0, 0), (0, 16 - payment_embed.shape[1])))
    ce_flat = jnp.reshape(currencies_embed, (-1,))
    pe_flat = jnp.reshape(pe_pad, (-1,))
    mesh = plsc.VectorSubcoreMesh(core_axis_name="c", subcore_axis_name="s",
                                  num_cores=NUM_CORES)
    k = functools.partial(_edge_embed_kernel, B, CHUNK)
    slot_types = []
    for _ in range(NSLOT):
        slot_types += [
            pltpu.VMEM((CHUNK,), jnp.int32),
            pltpu.VMEM((CHUNK,), jnp.int32),
            pltpu.VMEM((CHUNK,), jnp.int32),
            pltpu.VMEM((CHUNK, 16), jnp.float32),
            pltpu.VMEM((CHUNK, 56), jnp.float32),
        ]
    run = pl.kernel(
        k,
        out_type=jax.ShapeDtypeStruct((B, D_out), jnp.float32),
        mesh=mesh,
        compiler_params=pltpu.CompilerParams(needs_layout_passes=False),
        scratch_types=[
            pltpu.VMEM((ce_flat.shape[0],), jnp.float32),
            pltpu.VMEM((pe_flat.shape[0],), jnp.float32),
        ] + slot_types,
    )
    return run(payment_curr, receiving_curr, payment_method, numericals,
               ce_flat, pe_flat)
